# baseline (speedup 1.0000x reference)
import jax
import jax.numpy as jnp
from jax.experimental import pallas as pl
from jax.experimental.pallas import tpu as pltpu

D = 768
NH = 12
DH = 64
NE = 8
TK = 2
DHID = 1536
S = 2048
AEPS = 1e-6
FEPS = 1e-6

BR = 256
KS = 512
BR2 = 2048
BH = 768


def _block_body(sp_ref, x_ref, wq_ref, wqs_ref, wk_ref, wks_ref, wv_ref,
                cos_ref, sin_ref, anw_ref,
                wo_ref, fnw_ref, rw_ref,
                h_ref, hn_ref, g_ref, i_ref, acc_ref, ks, vs):
    r = pl.program_id(0)
    x = x_ref[...]
    xn = x * jax.lax.rsqrt(jnp.mean(x * x, axis=1, keepdims=True) + AEPS) * anw_ref[...]
    cs = jnp.concatenate([cos_ref[...]] * NH, axis=1)
    sn = jnp.concatenate([sin_ref[...]] * NH, axis=1)
    q = ((jnp.dot(xn, wq_ref[...], preferred_element_type=jnp.float32) * cs
          + jnp.dot(xn, wqs_ref[...], preferred_element_type=jnp.float32) * sn)
         .astype(jnp.bfloat16))
    k = ((jnp.dot(xn, wk_ref[...], preferred_element_type=jnp.float32) * cs
          + jnp.dot(xn, wks_ref[...], preferred_element_type=jnp.float32) * sn)
         .astype(jnp.bfloat16))
    v = jnp.dot(xn, wv_ref[...],
                preferred_element_type=jnp.float32).astype(jnp.bfloat16)

    @pl.when(r == 0)
    def _():
        vs[...] = jnp.zeros((S, D), jnp.bfloat16)

    ks[pl.ds(r * BR, BR), :] = k
    vs[pl.ds(r * BR, BR), :] = v

    rows = r * BR + jax.lax.broadcasted_iota(jnp.int32, (BR, S), 0) + sp_ref[0]
    cols = jax.lax.broadcasted_iota(jnp.int32, (BR, S), 1)
    mask = cols <= rows
    outs = []
    for h in range(NH):
        qh = q[:, h * DH:(h + 1) * DH]
        kh = ks[:, h * DH:(h + 1) * DH]
        s = jax.lax.dot_general(qh, kh, (((1,), (1,)), ((), ())),
                                preferred_element_type=jnp.float32) * 0.125
        s = jnp.where(mask, s, -1e9)
        m = jnp.max(s, axis=1, keepdims=True)
        p = jnp.exp(s - m)
        l = jnp.sum(p, axis=1, keepdims=True)
        pv = jnp.dot(p.astype(jnp.bfloat16), vs[:, h * DH:(h + 1) * DH],
                     preferred_element_type=jnp.float32)
        outs.append((pv / l).astype(jnp.bfloat16))

    attn = jnp.concatenate(outs, axis=1)
    h = x + jnp.dot(attn, wo_ref[...], preferred_element_type=jnp.float32)
    h_ref[...] = h
    hn = h * jax.lax.rsqrt(jnp.mean(h * h, axis=1, keepdims=True) + FEPS) * fnw_ref[...]
    hn_ref[...] = hn.astype(jnp.bfloat16)
    logits = jnp.dot(hn, rw_ref[...], preferred_element_type=jnp.float32)
    lane = jax.lax.broadcasted_iota(jnp.int32, (BR, 128), 1)
    valid = lane < NE
    logits = jnp.where(valid, logits, -jnp.inf)
    m = jnp.max(logits, axis=1, keepdims=True)
    e = jnp.exp(logits - m)
    probs = e / jnp.sum(e, axis=1, keepdims=True)
    v1 = jnp.max(probs, axis=1, keepdims=True)
    i1 = jnp.min(jnp.where(probs == v1, lane, NE), axis=1, keepdims=True)
    p2 = jnp.where(lane == i1, -1.0, probs)
    v2 = jnp.max(p2, axis=1, keepdims=True)
    i2 = jnp.min(jnp.where(p2 == v2, lane, NE), axis=1, keepdims=True)
    gs = v1 + v2
    col0 = lane == 0
    col1 = lane == 1
    g_ref[...] = jnp.where(col0, v1 / gs, 0.0) + jnp.where(col1, v2 / gs, 0.0)
    i_ref[...] = jnp.where(col0, i1, 0) + jnp.where(col1, i2, 0)

    @pl.when(r == 0)
    def _():
        acc_ref[...] = jnp.zeros_like(acc_ref)

    psum = jnp.sum(probs, axis=0, keepdims=True)
    dsum = jnp.sum((lane == i1).astype(jnp.float32)
                   + (lane == i2).astype(jnp.float32), axis=0, keepdims=True)
    acc_ref[0:1, :] += psum
    acc_ref[1:2, :] += dsum


def _moe_body(hn_ref, h_ref, g_ref, i_ref, w1_ref, w2_ref, o_ref):
    e = pl.program_id(1)
    hh = pl.program_id(2)

    @pl.when((e == 0) & (hh == 0))
    def _():
        o_ref[...] = h_ref[...]

    t = jnp.maximum(jnp.dot(hn_ref[...], w1_ref[0],
                            preferred_element_type=jnp.float32), 0.0)
    y = jnp.dot(t, w2_ref[0], preferred_element_type=jnp.float32)
    g1 = g_ref[:, 0:1]
    g2 = g_ref[:, 1:2]
    i1 = i_ref[:, 0:1]
    i2 = i_ref[:, 1:2]
    ge = jnp.where(i1 == e, g1, 0.0) + jnp.where(i2 == e, g2, 0.0)
    o_ref[...] += ge * y


def _swapw(w):
    wr = w.reshape(D, D // 2, 2)
    return jnp.stack([-wr[:, :, 1], wr[:, :, 0]], axis=-1).reshape(D, D)


def kernel(x, freqs_complex, start_pos, attn_norm_w, ffn_norm_w,
           wq, wk, wv, wo, router_w, w1, w2):
    xf = x.reshape(S, D)
    cos64 = jnp.repeat(jnp.cos(freqs_complex), 2, axis=1)
    sin64 = jnp.repeat(jnp.sin(freqs_complex), 2, axis=1)
    wqs = _swapw(wq)
    wks = _swapw(wk)
    anw = attn_norm_w.reshape(1, D)
    fnw = ffn_norm_w.reshape(1, D)
    rwp = jnp.pad(router_w, ((0, 0), (0, 128 - NE)))
    sp = jnp.asarray(start_pos, jnp.int32).reshape(1)

    h, hn, g, idx, acc = pl.pallas_call(
        _block_body,
        grid=(S // BR,),
        in_specs=[
            pl.BlockSpec(memory_space=pltpu.SMEM),
            pl.BlockSpec((BR, D), lambda r: (r, 0)),
            pl.BlockSpec((D, D), lambda r: (0, 0)),
            pl.BlockSpec((D, D), lambda r: (0, 0)),
            pl.BlockSpec((D, D), lambda r: (0, 0)),
            pl.BlockSpec((D, D), lambda r: (0, 0)),
            pl.BlockSpec((D, D), lambda r: (0, 0)),
            pl.BlockSpec((BR, DH), lambda r: (r, 0)),
            pl.BlockSpec((BR, DH), lambda r: (r, 0)),
            pl.BlockSpec((1, D), lambda r: (0, 0)),
            pl.BlockSpec((D, D), lambda r: (0, 0)),
            pl.BlockSpec((1, D), lambda r: (0, 0)),
            pl.BlockSpec((D, 128), lambda r: (0, 0)),
        ],
        out_specs=[
            pl.BlockSpec((BR, D), lambda r: (r, 0)),
            pl.BlockSpec((BR, D), lambda r: (r, 0)),
            pl.BlockSpec((BR, 128), lambda r: (r, 0)),
            pl.BlockSpec((BR, 128), lambda r: (r, 0)),
            pl.BlockSpec((8, 128), lambda r: (0, 0)),
        ],
        out_shape=[
            jax.ShapeDtypeStruct((S, D), jnp.float32),
            jax.ShapeDtypeStruct((S, D), jnp.bfloat16),
            jax.ShapeDtypeStruct((S, 128), jnp.float32),
            jax.ShapeDtypeStruct((S, 128), jnp.int32),
            jax.ShapeDtypeStruct((8, 128), jnp.float32),
        ],
        scratch_shapes=[
            pltpu.VMEM((S, D), jnp.bfloat16),
            pltpu.VMEM((S, D), jnp.bfloat16),
        ],
    )(sp, xf, wq, wqs, wk, wks, wv, cos64, sin64, anw, wo, fnw, rwp)

    out = pl.pallas_call(
        _moe_body,
        grid=(S // BR2, NE, DHID // BH),
        in_specs=[
            pl.BlockSpec((BR2, D), lambda r, e, hh: (r, 0)),
            pl.BlockSpec((BR2, D), lambda r, e, hh: (r, 0)),
            pl.BlockSpec((BR2, 128), lambda r, e, hh: (r, 0)),
            pl.BlockSpec((BR2, 128), lambda r, e, hh: (r, 0)),
            pl.BlockSpec((1, D, BH), lambda r, e, hh: (e, 0, hh)),
            pl.BlockSpec((1, BH, D), lambda r, e, hh: (e, hh, 0)),
        ],
        out_specs=pl.BlockSpec((BR2, D), lambda r, e, hh: (r, 0)),
        out_shape=jax.ShapeDtypeStruct((S, D), jnp.float32),
    )(hn, h, g, idx, w1, w2)

    f = acc[1, :NE] / (S * TK)
    p = acc[0, :NE] / S
    loss = NE * jnp.sum(f * p)
    return (out.reshape(1, S, D), loss)


# speedup vs baseline: 1.0291x; 1.0291x over previous
import jax
import jax.numpy as jnp
from jax.experimental import pallas as pl
from jax.experimental.pallas import tpu as pltpu

D = 768
NH = 12
DH = 64
NE = 8
TK = 2
DHID = 1536
S = 2048
AEPS = 1e-6
FEPS = 1e-6

BR = 256
KS = 512
BR2 = 2048
BH = 512


def _block_body(sp_ref, x_ref, wq_ref, wqs_ref, wk_ref, wks_ref, wv_ref,
                cos_ref, sin_ref, anw_ref,
                wo_ref, fnw_ref, rw_ref,
                h_ref, hn_ref, g_ref, i_ref, acc_ref, ks, vs):
    r = pl.program_id(0)
    x = x_ref[...]
    xn = x * jax.lax.rsqrt(jnp.mean(x * x, axis=1, keepdims=True) + AEPS) * anw_ref[...]
    cs = jnp.concatenate([cos_ref[...]] * NH, axis=1)
    sn = jnp.concatenate([sin_ref[...]] * NH, axis=1)
    q = ((jnp.dot(xn, wq_ref[...], preferred_element_type=jnp.float32) * cs
          + jnp.dot(xn, wqs_ref[...], preferred_element_type=jnp.float32) * sn)
         .astype(jnp.bfloat16))
    k = ((jnp.dot(xn, wk_ref[...], preferred_element_type=jnp.float32) * cs
          + jnp.dot(xn, wks_ref[...], preferred_element_type=jnp.float32) * sn)
         .astype(jnp.bfloat16))
    v = jnp.dot(xn, wv_ref[...],
                preferred_element_type=jnp.float32).astype(jnp.bfloat16)

    @pl.when(r == 0)
    def _():
        vs[...] = jnp.zeros((S, D), jnp.bfloat16)

    ks[pl.ds(r * BR, BR), :] = k
    vs[pl.ds(r * BR, BR), :] = v

    rows = r * BR + jax.lax.broadcasted_iota(jnp.int32, (BR, S), 0) + sp_ref[0]
    cols = jax.lax.broadcasted_iota(jnp.int32, (BR, S), 1)
    mask = cols <= rows
    outs = []
    for h in range(NH):
        qh = q[:, h * DH:(h + 1) * DH]
        kh = ks[:, h * DH:(h + 1) * DH]
        s = jax.lax.dot_general(qh, kh, (((1,), (1,)), ((), ())),
                                preferred_element_type=jnp.float32) * 0.125
        s = jnp.where(mask, s, -1e9)
        m = jnp.max(s, axis=1, keepdims=True)
        p = jnp.exp(s - m)
        l = jnp.sum(p, axis=1, keepdims=True)
        pv = jnp.dot(p.astype(jnp.bfloat16), vs[:, h * DH:(h + 1) * DH],
                     preferred_element_type=jnp.float32)
        outs.append((pv / l).astype(jnp.bfloat16))

    attn = jnp.concatenate(outs, axis=1)
    h = x + jnp.dot(attn, wo_ref[...], preferred_element_type=jnp.float32)
    h_ref[...] = h
    hn = h * jax.lax.rsqrt(jnp.mean(h * h, axis=1, keepdims=True) + FEPS) * fnw_ref[...]
    hn_ref[...] = hn.astype(jnp.bfloat16)
    logits = jnp.dot(hn, rw_ref[...], preferred_element_type=jnp.float32)
    lane = jax.lax.broadcasted_iota(jnp.int32, (BR, 128), 1)
    valid = lane < NE
    logits = jnp.where(valid, logits, -jnp.inf)
    m = jnp.max(logits, axis=1, keepdims=True)
    e = jnp.exp(logits - m)
    probs = e / jnp.sum(e, axis=1, keepdims=True)
    v1 = jnp.max(probs, axis=1, keepdims=True)
    i1 = jnp.min(jnp.where(probs == v1, lane, NE), axis=1, keepdims=True)
    p2 = jnp.where(lane == i1, -1.0, probs)
    v2 = jnp.max(p2, axis=1, keepdims=True)
    i2 = jnp.min(jnp.where(p2 == v2, lane, NE), axis=1, keepdims=True)
    gs = v1 + v2
    col0 = lane == 0
    col1 = lane == 1
    g_ref[...] = jnp.where(col0, v1 / gs, 0.0) + jnp.where(col1, v2 / gs, 0.0)
    i_ref[...] = jnp.where(col0, i1, 0) + jnp.where(col1, i2, 0)

    @pl.when(r == 0)
    def _():
        acc_ref[...] = jnp.zeros_like(acc_ref)

    psum = jnp.sum(probs, axis=0, keepdims=True)
    dsum = jnp.sum((lane == i1).astype(jnp.float32)
                   + (lane == i2).astype(jnp.float32), axis=0, keepdims=True)
    acc_ref[0:1, :] += psum
    acc_ref[1:2, :] += dsum


def _moe_body(hn_ref, h_ref, g_ref, i_ref, w1_ref, w2_ref, o_ref):
    e = pl.program_id(1)
    hh = pl.program_id(2)

    @pl.when((e == 0) & (hh == 0))
    def _():
        o_ref[...] = h_ref[...]

    t = jnp.maximum(jnp.dot(hn_ref[...], w1_ref[0],
                            preferred_element_type=jnp.float32), 0.0)
    y = jnp.dot(t, w2_ref[0], preferred_element_type=jnp.float32)
    g1 = g_ref[:, 0:1]
    g2 = g_ref[:, 1:2]
    i1 = i_ref[:, 0:1]
    i2 = i_ref[:, 1:2]
    ge = jnp.where(i1 == e, g1, 0.0) + jnp.where(i2 == e, g2, 0.0)
    o_ref[...] += ge * y


def _swapw(w):
    wr = w.reshape(D, D // 2, 2)
    return jnp.stack([-wr[:, :, 1], wr[:, :, 0]], axis=-1).reshape(D, D)


def kernel(x, freqs_complex, start_pos, attn_norm_w, ffn_norm_w,
           wq, wk, wv, wo, router_w, w1, w2):
    xf = x.reshape(S, D)
    cos64 = jnp.repeat(jnp.cos(freqs_complex), 2, axis=1)
    sin64 = jnp.repeat(jnp.sin(freqs_complex), 2, axis=1)
    wqs = _swapw(wq)
    wks = _swapw(wk)
    anw = attn_norm_w.reshape(1, D)
    fnw = ffn_norm_w.reshape(1, D)
    rwp = jnp.pad(router_w, ((0, 0), (0, 128 - NE)))
    sp = jnp.asarray(start_pos, jnp.int32).reshape(1)

    h, hn, g, idx, acc = pl.pallas_call(
        _block_body,
        grid=(S // BR,),
        in_specs=[
            pl.BlockSpec(memory_space=pltpu.SMEM),
            pl.BlockSpec((BR, D), lambda r: (r, 0)),
            pl.BlockSpec((D, D), lambda r: (0, 0)),
            pl.BlockSpec((D, D), lambda r: (0, 0)),
            pl.BlockSpec((D, D), lambda r: (0, 0)),
            pl.BlockSpec((D, D), lambda r: (0, 0)),
            pl.BlockSpec((D, D), lambda r: (0, 0)),
            pl.BlockSpec((BR, DH), lambda r: (r, 0)),
            pl.BlockSpec((BR, DH), lambda r: (r, 0)),
            pl.BlockSpec((1, D), lambda r: (0, 0)),
            pl.BlockSpec((D, D), lambda r: (0, 0)),
            pl.BlockSpec((1, D), lambda r: (0, 0)),
            pl.BlockSpec((D, 128), lambda r: (0, 0)),
        ],
        out_specs=[
            pl.BlockSpec((BR, D), lambda r: (r, 0)),
            pl.BlockSpec((BR, D), lambda r: (r, 0)),
            pl.BlockSpec((BR, 128), lambda r: (r, 0)),
            pl.BlockSpec((BR, 128), lambda r: (r, 0)),
            pl.BlockSpec((8, 128), lambda r: (0, 0)),
        ],
        out_shape=[
            jax.ShapeDtypeStruct((S, D), jnp.float32),
            jax.ShapeDtypeStruct((S, D), jnp.bfloat16),
            jax.ShapeDtypeStruct((S, 128), jnp.float32),
            jax.ShapeDtypeStruct((S, 128), jnp.int32),
            jax.ShapeDtypeStruct((8, 128), jnp.float32),
        ],
        scratch_shapes=[
            pltpu.VMEM((S, D), jnp.bfloat16),
            pltpu.VMEM((S, D), jnp.bfloat16),
        ],
    )(sp, xf, wq, wqs, wk, wks, wv, cos64, sin64, anw, wo, fnw, rwp)

    out = pl.pallas_call(
        _moe_body,
        grid=(S // BR2, NE, DHID // BH),
        in_specs=[
            pl.BlockSpec((BR2, D), lambda r, e, hh: (r, 0)),
            pl.BlockSpec((BR2, D), lambda r, e, hh: (r, 0)),
            pl.BlockSpec((BR2, 128), lambda r, e, hh: (r, 0)),
            pl.BlockSpec((BR2, 128), lambda r, e, hh: (r, 0)),
            pl.BlockSpec((1, D, BH), lambda r, e, hh: (e, 0, hh)),
            pl.BlockSpec((1, BH, D), lambda r, e, hh: (e, hh, 0)),
        ],
        out_specs=pl.BlockSpec((BR2, D), lambda r, e, hh: (r, 0)),
        out_shape=jax.ShapeDtypeStruct((S, D), jnp.float32),
    )(hn, h, g, idx, w1, w2)

    f = acc[1, :NE] / (S * TK)
    p = acc[0, :NE] / S
    loss = NE * jnp.sum(f * p)
    return (out.reshape(1, S, D), loss)
